# baseline (speedup 1.0000x reference)
import functools

import jax
import jax.numpy as jnp
from jax import lax
from jax.experimental import pallas as pl
from jax.experimental.pallas import tpu as pltpu
from jax.experimental.pallas import tpu_sc as plsc

VOCAB = 1000000
D = 64
B_TOTAL = 4096 * 200
NW = 32
B_PER_W = B_TOTAL // NW
IDX_W = 128
G = 8
CHUNK = G * IDX_W
N_CHUNKS = B_PER_W // CHUNK
IDX_ROWS_PER_W = B_PER_W // IDX_W

_mesh = plsc.VectorSubcoreMesh(core_axis_name="c", subcore_axis_name="s")


@functools.partial(
    pl.kernel,
    mesh=_mesh,
    out_type=jax.ShapeDtypeStruct((B_TOTAL, D), jnp.float32),
    scratch_types=[
        pltpu.VMEM((G, IDX_W), jnp.int32),
        pltpu.VMEM((CHUNK, D), jnp.float32),
        pltpu.SemaphoreType.DMA,
    ],
    compiler_params=pltpu.CompilerParams(use_tc_tiling_on_sc=False),
)
def _embed_gather(idx_hbm, table_hbm, out_hbm, idx_v, rows_v, sem):
    wid = lax.axis_index("s") * 2 + lax.axis_index("c")
    row0 = wid * IDX_ROWS_PER_W

    def body(j, carry):
        pltpu.sync_copy(idx_hbm.at[pl.ds(row0 + j * G, G)], idx_v)
        copies = []
        for g in range(G):
            copies.append(
                pltpu.async_copy(
                    table_hbm.at[idx_v.at[g]],
                    rows_v.at[pl.ds(g * IDX_W, IDX_W)],
                    sem,
                )
            )
        for c in copies:
            c.wait()
        out0 = (row0 + j * G) * IDX_W
        pltpu.sync_copy(rows_v, out_hbm.at[pl.ds(out0, CHUNK)])
        return carry

    lax.fori_loop(0, N_CHUNKS, body, 0)


def kernel(idx_texts, table):
    idx_flat = idx_texts.reshape(B_TOTAL // IDX_W, IDX_W).astype(jnp.int32)
    out = _embed_gather(idx_flat, table)
    return out.reshape(idx_texts.shape + (D,))


# speedup vs baseline: 1.0117x; 1.0117x over previous
import functools

import jax
import jax.numpy as jnp
from jax import lax
from jax.experimental import pallas as pl
from jax.experimental.pallas import tpu as pltpu
from jax.experimental.pallas import tpu_sc as plsc

VOCAB = 1000000
D = 64
B_TOTAL = 4096 * 200
NW = 32
B_PER_W = B_TOTAL // NW
IDX_W = 128
IDX_ROWS_PER_W = B_PER_W // IDX_W
G = 5
CHUNK = G * IDX_W
N_CHUNKS = B_PER_W // CHUNK
M_PAIRS = N_CHUNKS // 2

_mesh = plsc.VectorSubcoreMesh(core_axis_name="c", subcore_axis_name="s")


@functools.partial(
    pl.kernel,
    mesh=_mesh,
    out_type=jax.ShapeDtypeStruct((B_TOTAL, D), jnp.float32),
    scratch_types=[
        pltpu.VMEM((IDX_ROWS_PER_W, IDX_W), jnp.int32),
        pltpu.VMEM((CHUNK, D), jnp.float32),
        pltpu.VMEM((CHUNK, D), jnp.float32),
        pltpu.SemaphoreType.DMA,
        pltpu.SemaphoreType.DMA,
        pltpu.SemaphoreType.DMA,
        pltpu.SemaphoreType.DMA,
    ],
    compiler_params=pltpu.CompilerParams(use_tc_tiling_on_sc=False),
)
def _embed_gather(idx_hbm, table_hbm, out_hbm, idx_all, rows0, rows1,
                  sem_g0, sem_g1, sem_s0, sem_s1):
    wid = lax.axis_index("s") * 2 + lax.axis_index("c")
    idx_row0 = wid * IDX_ROWS_PER_W
    out_row0 = wid * B_PER_W

    pltpu.sync_copy(idx_hbm.at[pl.ds(idx_row0, IDX_ROWS_PER_W)], idx_all)

    def fire_gathers(j, rows, sem):
        r0 = j * G
        return [
            pltpu.async_copy(
                table_hbm.at[idx_all.at[r0 + g]],
                rows.at[pl.ds(g * IDX_W, IDX_W)],
                sem,
            )
            for g in range(G)
        ]

    def fire_store(j, rows, sem):
        return pltpu.async_copy(
            rows, out_hbm.at[pl.ds(out_row0 + j * CHUNK, CHUNK)], sem)

    def wait_store(j, rows, sem):
        pltpu.make_async_copy(
            rows, out_hbm.at[pl.ds(out_row0 + j * CHUNK, CHUNK)], sem).wait()

    def pair(m, first, last):
        j0 = 2 * m
        j1 = j0 + 1
        if not first:
            wait_store(j0 - 2, rows0, sem_s0)
        g0 = fire_gathers(j0, rows0, sem_g0)
        if not first:
            wait_store(j1 - 2, rows1, sem_s1)
        g1 = fire_gathers(j1, rows1, sem_g1)
        for c in g0:
            c.wait()
        fire_store(j0, rows0, sem_s0)
        for c in g1:
            c.wait()
        fire_store(j1, rows1, sem_s1)
        if last:
            wait_store(j0, rows0, sem_s0)
            wait_store(j1, rows1, sem_s1)

    pair(0, True, False)

    def body(m, carry):
        pair(m, False, False)
        return carry

    lax.fori_loop(1, M_PAIRS - 1, body, 0)
    pair(M_PAIRS - 1, False, True)


def kernel(idx_texts, table):
    idx_flat = idx_texts.reshape(B_TOTAL // IDX_W, IDX_W).astype(jnp.int32)
    out = _embed_gather(idx_flat, table)
    return out.reshape(idx_texts.shape + (D,))
